# Initial kernel scaffold; baseline (speedup 1.0000x reference)
import jax
import jax.numpy as jnp
from jax.experimental import pallas as pl


def kernel(edge_index, emb, gconv_params, dense_params):
    raise NotImplementedError("write your pallas kernel here")


# speedup vs baseline: 19.1650x; 19.1650x over previous
import functools

import jax
import jax.numpy as jnp
from jax import lax
from jax.experimental import pallas as pl
from jax.experimental.pallas import tpu as pltpu
from jax.experimental.pallas import tpu_sc as plsc

N = 100000
E = 1600000
K = 5
NC = 2
NS = 16
NW = NC * NS

NP = 100352
NPW = NP // NS
NPP = NP * 16 // 128

ECHUNK = 128
GKS = 8
S_GROUPS = 50
EP = 1638400
GKH = 16
H_GROUPS = 25
HP = 3276800
EH = HP // 2
GHP = EH * 16 // 128

SM_CH = 2000
SM_NCH = 25


def _mesh():
    return plsc.VectorSubcoreMesh(core_axis_name="c", subcore_axis_name="s")


def _make_deg_kernel():
    @functools.partial(
        pl.kernel,
        out_type=jax.ShapeDtypeStruct((NC, NP, 16), jnp.float32),
        mesh=_mesh(),
        compiler_params=pltpu.CompilerParams(use_tc_tiling_on_sc=False),
        scratch_types=[
            pltpu.VMEM((ECHUNK, 16), jnp.float32),
            pltpu.VMEM((GKS, ECHUNK), jnp.int32),
            pltpu.VMEM_SHARED((NP, 16), jnp.float32),
        ],
    )
    def deg_kernel(row2_hbm, ones_hbm, z_hbm, out_hbm, ones_v, sidx_v, acc):
        cid = lax.axis_index("c")
        sid = lax.axis_index("s")
        wid = sid * NC + cid
        pltpu.sync_copy(ones_hbm, ones_v)
        pltpu.sync_copy(z_hbm, acc.at[pl.ds(sid * NPW, NPW)])
        plsc.subcore_barrier()
        base = wid * (S_GROUPS * GKS)

        def body(g, carry):
            cbase = base + g * GKS
            pltpu.sync_copy(row2_hbm.at[pl.ds(cbase, GKS)], sidx_v)
            for j in range(GKS):
                pltpu.sync_copy(ones_v, acc.at[sidx_v.at[j]], add=True)
            return carry

        lax.fori_loop(0, S_GROUPS, body, 0)
        plsc.subcore_barrier()
        pltpu.sync_copy(acc.at[pl.ds(sid * NPW, NPW)],
                        out_hbm.at[cid, pl.ds(sid * NPW, NPW)])

    return deg_kernel


def _make_s_kernel():
    C = 16

    @functools.partial(
        pl.kernel,
        out_type=jax.ShapeDtypeStruct((NC, NP, C), jnp.float32),
        mesh=_mesh(),
        compiler_params=pltpu.CompilerParams(use_tc_tiling_on_sc=False),
        scratch_types=[
            pltpu.VMEM((GKS * ECHUNK,), jnp.int32),
            pltpu.VMEM((GKS, ECHUNK), jnp.int32),
            pltpu.VMEM((GKS // 2, ECHUNK, C), jnp.float32),
            pltpu.VMEM((GKS // 2, ECHUNK, C), jnp.float32),
            pltpu.VMEM_SHARED((NP, C), jnp.float32),
            pltpu.SemaphoreType.DMA,
            pltpu.SemaphoreType.DMA,
            pltpu.SemaphoreType.DMA,
            pltpu.SemaphoreType.DMA,
        ],
    )
    def s_kernel(u_hbm, row_hbm, col2_hbm, z_hbm, out_hbm,
                 gidx_v, sidx_v, rows_a, rows_b, acc,
                 sem_ga, sem_gb, sem_sa, sem_sb):
        cid = lax.axis_index("c")
        sid = lax.axis_index("s")
        wid = sid * NC + cid
        pltpu.sync_copy(z_hbm, acc.at[pl.ds(sid * NPW, NPW)])
        plsc.subcore_barrier()
        base = wid * (S_GROUPS * GKS)
        H = GKS // 2

        def fire_gathers(rows_v, off, sem):
            return [
                pltpu.async_copy(
                    u_hbm.at[gidx_v.at[pl.ds((off + j) * ECHUNK, ECHUNK)]],
                    rows_v.at[j], sem)
                for j in range(H)
            ]

        def fire_scatters(rows_v, off, sem):
            return [
                pltpu.async_copy(rows_v.at[j], acc.at[sidx_v.at[off + j]],
                                 sem, add=True)
                for j in range(H)
            ]

        def drain(descs):
            for d in descs:
                d.wait()

        def drain_prev_scatters():
            drain([pltpu.make_async_copy(rows_a.at[j],
                                         acc.at[sidx_v.at[j]], sem_sa)
                   for j in range(H)])
            drain([pltpu.make_async_copy(rows_b.at[j],
                                         acc.at[sidx_v.at[H + j]], sem_sb)
                   for j in range(H)])

        def do_pair(p, drain_scatters):
            if drain_scatters:
                drain_prev_scatters()
            cbase = base + p * GKS
            pltpu.sync_copy(row_hbm.at[pl.ds(cbase * ECHUNK, GKS * ECHUNK)],
                            gidx_v)
            pltpu.sync_copy(col2_hbm.at[pl.ds(cbase, GKS)], sidx_v)
            ga = fire_gathers(rows_a, 0, sem_ga)
            gb = fire_gathers(rows_b, H, sem_gb)
            drain(ga)
            fire_scatters(rows_a, 0, sem_sa)
            drain(gb)
            fire_scatters(rows_b, H, sem_sb)

        do_pair(0, False)

        def body(p, carry):
            do_pair(p, True)
            return carry

        lax.fori_loop(1, S_GROUPS, body, 0)
        drain_prev_scatters()
        plsc.subcore_barrier()
        pltpu.sync_copy(acc.at[pl.ds(sid * NPW, NPW)],
                        out_hbm.at[cid, pl.ds(sid * NPW, NPW)])

    return s_kernel


def _make_gather_kernel():
    NCH = HP // ECHUNK

    @functools.partial(
        pl.kernel,
        out_type=jax.ShapeDtypeStruct((NCH, ECHUNK, 16), jnp.float32),
        mesh=_mesh(),
        compiler_params=pltpu.CompilerParams(use_tc_tiling_on_sc=False),
        scratch_types=[
            pltpu.VMEM((GKH * ECHUNK,), jnp.int32),
            pltpu.VMEM((GKH * ECHUNK,), jnp.int32),
            pltpu.VMEM((GKH, ECHUNK, 16), jnp.float32),
            pltpu.VMEM((GKH, ECHUNK, 16), jnp.float32),
            pltpu.SemaphoreType.DMA,
            pltpu.SemaphoreType.DMA,
        ],
    )
    def gather_kernel(h_hbm, idx_hbm, out_hbm,
                      gidx_a, gidx_b, rows_a, rows_b, sem_a, sem_b):
        cid = lax.axis_index("c")
        sid = lax.axis_index("s")
        wid = sid * NC + cid
        base = wid * (H_GROUPS * 2 * GKH)

        def half(cbase, gidx_v, rows_v, sem):
            pltpu.sync_copy(idx_hbm.at[pl.ds(cbase * ECHUNK, GKH * ECHUNK)],
                            gidx_v)
            return [
                pltpu.async_copy(
                    h_hbm.at[gidx_v.at[pl.ds(j * ECHUNK, ECHUNK)]],
                    rows_v.at[j], sem)
                for j in range(GKH)
            ]

        def body(g, carry):
            ca = base + g * (2 * GKH)
            cb = ca + GKH
            da = half(ca, gidx_a, rows_a, sem_a)
            db = half(cb, gidx_b, rows_b, sem_b)
            for d in da:
                d.wait()
            pltpu.sync_copy(rows_a, out_hbm.at[pl.ds(ca, GKH)])
            for d in db:
                d.wait()
            pltpu.sync_copy(rows_b, out_hbm.at[pl.ds(cb, GKH)])
            return carry

        lax.fori_loop(0, H_GROUPS, body, 0)

    return gather_kernel


def _make_softmax3_kernel():
    @functools.partial(
        pl.kernel,
        out_type=jax.ShapeDtypeStruct((3, E), jnp.float32),
        mesh=_mesh(),
        compiler_params=pltpu.CompilerParams(use_tc_tiling_on_sc=False,
                                             needs_layout_passes=False),
        scratch_types=[
            pltpu.VMEM((SM_CH * 16,), jnp.float32),
            pltpu.VMEM((SM_CH,), jnp.float32),
            pltpu.VMEM((SM_CH,), jnp.float32),
            pltpu.VMEM((SM_CH,), jnp.float32),
        ],
    )
    def softmax3_kernel(x_hbm, out_hbm, in_v, o0_v, o1_v, o2_v):
        cid = lax.axis_index("c")
        sid = lax.axis_index("s")
        wid = sid * NC + cid
        tile_e0 = wid * (SM_NCH * SM_CH)
        lanes = lax.iota(jnp.int32, 16)
        lane_off = 128 * (lanes // 8) + 3 * (lanes - 8 * (lanes // 8))

        def chunk(ch, carry):
            e0 = tile_e0 + ch * SM_CH
            pltpu.sync_copy(x_hbm.at[pl.ds(e0 * 16, SM_CH * 16)], in_v)

            def group(g, c2):
                bidx = 256 * g + lane_off
                x0 = plsc.load_gather(in_v, [bidx])
                x1 = plsc.load_gather(in_v, [bidx + 1])
                x2 = plsc.load_gather(in_v, [bidx + 2])
                m = jnp.maximum(jnp.maximum(x0, x1), x2)
                e0_ = jnp.exp(x0 - m)
                e1_ = jnp.exp(x1 - m)
                e2_ = jnp.exp(x2 - m)
                s = e0_ + e1_ + e2_
                o0_v[pl.ds(g * 16, 16)] = e0_ / s
                o1_v[pl.ds(g * 16, 16)] = e1_ / s
                o2_v[pl.ds(g * 16, 16)] = e2_ / s
                return c2

            lax.fori_loop(0, SM_CH // 16, group, 0)
            pltpu.sync_copy(o0_v, out_hbm.at[0, pl.ds(e0, SM_CH)])
            pltpu.sync_copy(o1_v, out_hbm.at[1, pl.ds(e0, SM_CH)])
            pltpu.sync_copy(o2_v, out_hbm.at[2, pl.ds(e0, SM_CH)])
            return carry

        lax.fori_loop(0, SM_NCH, chunk, 0)

    return softmax3_kernel


_RB = 2048
_PB = 1792
_HBP = 2048


def _kron(n, w):
    return jnp.kron(jnp.eye(n, dtype=jnp.float32), w)


def _tc_dinv(d0, d1):
    def body(d0_ref, d1_ref, dinv_ref):
        deg = d0_ref[...] + d1_ref[...]
        dinv_ref[...] = jnp.where(deg > 0,
                                  lax.rsqrt(jnp.maximum(deg, 1e-12)),
                                  jnp.zeros_like(deg))

    spec = pl.BlockSpec((_PB, 128), lambda i: (i, 0))
    return pl.pallas_call(
        body,
        grid=(NPP // _PB,),
        in_specs=[spec, spec],
        out_specs=spec,
        out_shape=jax.ShapeDtypeStruct((NPP, 128), jnp.float32),
    )(d0, d1)


PK_CH = 784
PK_NCH = 4


def _make_pack_kernel(Cin):
    @functools.partial(
        pl.kernel,
        out_type=[jax.ShapeDtypeStruct((NP * 16,), jnp.float32)] * 4,
        mesh=_mesh(),
        compiler_params=pltpu.CompilerParams(use_tc_tiling_on_sc=False,
                                             needs_layout_passes=False),
        scratch_types=[
            pltpu.VMEM(((Cin + 1) * PK_CH,), jnp.float32),
            pltpu.VMEM((PK_CH * 16,), jnp.float32),
            pltpu.VMEM((PK_CH * 16,), jnp.float32),
            pltpu.VMEM((PK_CH * 16,), jnp.float32),
            pltpu.VMEM((PK_CH * 16,), jnp.float32),
            pltpu.VMEM((PK_CH * 16,), jnp.float32),
        ],
    )
    def pack_kernel(embT_hbm, dinvf_hbm, t0a_hbm, t0b_hbm, ua_hbm, ub_hbm,
                    in_v, dv_v, oa_v, ob_v, oua_v, oub_v):
        cid = lax.axis_index("c")
        sid = lax.axis_index("s")
        wid = sid * NC + cid
        n_base = wid * (PK_NCH * PK_CH)
        lanes = lax.iota(jnp.int32, 16)
        ib_rows = jnp.where(lanes < Cin - 16, (16 + lanes) * PK_CH,
                            Cin * PK_CH)
        for i in range(PK_CH // 16):
            in_v[pl.ds(Cin * PK_CH + i * 16, 16)] = jnp.zeros((16,),
                                                              jnp.float32)

        def chunk(ch, carry):
            n0 = n_base + ch * PK_CH
            for f in range(Cin):
                pltpu.sync_copy(embT_hbm.at[f, pl.ds(n0, PK_CH)],
                                in_v.at[pl.ds(f * PK_CH, PK_CH)])
            pltpu.sync_copy(dinvf_hbm.at[pl.ds(n0 * 16, PK_CH * 16)], dv_v)

            def node(dn, c2):
                xa = plsc.load_gather(in_v, [lanes * PK_CH + dn])
                xb = plsc.load_gather(in_v, [ib_rows + dn])
                dv = dv_v[pl.ds(dn * 16, 16)]
                oa_v[pl.ds(dn * 16, 16)] = xa
                ob_v[pl.ds(dn * 16, 16)] = xb
                oua_v[pl.ds(dn * 16, 16)] = xa * dv
                oub_v[pl.ds(dn * 16, 16)] = xb * dv
                return c2

            lax.fori_loop(0, PK_CH, node, 0)
            pltpu.sync_copy(oa_v, t0a_hbm.at[pl.ds(n0 * 16, PK_CH * 16)])
            pltpu.sync_copy(ob_v, t0b_hbm.at[pl.ds(n0 * 16, PK_CH * 16)])
            pltpu.sync_copy(oua_v, ua_hbm.at[pl.ds(n0 * 16, PK_CH * 16)])
            pltpu.sync_copy(oub_v, ub_hbm.at[pl.ds(n0 * 16, PK_CH * 16)])
            return carry

        lax.fori_loop(0, PK_NCH, chunk, 0)

    return pack_kernel


def _tc_cheb(ss, dinv, tprevs, alpha, beta):
    ns = len(ss)

    def body(*refs):
        ins = refs[:2 * ns]
        dv_ref = refs[2 * ns]
        tps = refs[2 * ns + 1:3 * ns + 1]
        touts = refs[3 * ns + 1:4 * ns + 1]
        uouts = refs[4 * ns + 1:]
        dv = dv_ref[...]
        for s in range(ns):
            t = (alpha * dv * (ins[2 * s][...] + ins[2 * s + 1][...])
                 + beta * tps[s][...])
            touts[s][...] = t
            uouts[s][...] = dv * t

    grid = NPP // _PB
    spec = pl.BlockSpec((_PB, 128), lambda i: (i, 0))
    flat = []
    for s0, s1 in ss:
        flat += [s0, s1]
    outs = pl.pallas_call(
        body,
        grid=(grid,),
        in_specs=[spec] * (2 * ns) + [spec] + [spec] * ns,
        out_specs=[spec] * (2 * ns),
        out_shape=[jax.ShapeDtypeStruct((NPP, 128), jnp.float32)] * (2 * ns),
    )(*flat, dinv, *tprevs)
    return outs[:ns], outs[ns:]


def _tc_gru(Ts_streams, Kz, Kh, bz, bh, dinv):
    ns = len(Ts_streams)

    def body(*refs):
        t_refs = refs[:ns * K]
        kz_refs = refs[ns * K:2 * ns * K]
        kh_refs = refs[2 * ns * K:3 * ns * K]
        bz_ref = refs[3 * ns * K]
        bh_ref = refs[3 * ns * K + 1]
        dv_ref = refs[3 * ns * K + 2]
        h_ref, u_ref = refs[-2:]
        Oz = jnp.broadcast_to(bz_ref[...], (_PB, 128))
        Oh = jnp.broadcast_to(bh_ref[...], (_PB, 128))
        for i in range(ns * K):
            t = t_refs[i][...]
            Oz = Oz + jnp.dot(t, kz_refs[i][...],
                              preferred_element_type=jnp.float32)
            Oh = Oh + jnp.dot(t, kh_refs[i][...],
                              preferred_element_type=jnp.float32)
        Z = jax.nn.sigmoid(Oz)
        Ht = jnp.tanh(Oh)
        h = jax.nn.relu((1.0 - Z) * Ht)
        h_ref[...] = h
        u_ref[...] = dv_ref[...] * h

    grid = NPP // _PB
    spec = pl.BlockSpec((_PB, 128), lambda i: (i, 0))
    wspec = pl.BlockSpec((128, 128), lambda i: (0, 0))
    bspec = pl.BlockSpec((1, 128), lambda i: (0, 0))
    flat_t = [t for Ts in Ts_streams for t in Ts]
    return pl.pallas_call(
        body,
        grid=(grid,),
        in_specs=([spec] * (ns * K) + [wspec] * (2 * ns * K)
                  + [bspec, bspec, spec]),
        out_specs=[spec, spec],
        out_shape=[jax.ShapeDtypeStruct((NPP, 128), jnp.float32)] * 2,
    )(*flat_t, *Kz, *Kh, bz, bh, dinv)


def _tc_head(ghp, gap, Ws, bs):
    K1h = _kron(8, Ws[0][:16, :])
    K1a = _kron(8, Ws[0][16:, :])
    Kmid = [_kron(8, w) for w in Ws[1:]]
    bsp = [jnp.tile(b, 8)[None, :] for b in bs]

    def body(gh_ref, ga_ref, k1h_ref, k1a_ref, k2, k3, k4, k5, k6,
             b1, b2, b3, b4, b5, b6, o_ref):
        x = jax.nn.relu(
            jnp.dot(gh_ref[...], k1h_ref[...],
                    preferred_element_type=jnp.float32)
            + jnp.dot(ga_ref[...], k1a_ref[...],
                      preferred_element_type=jnp.float32)
            + b1[...])
        for kr, br in ((k2, b2), (k3, b3), (k4, b4), (k5, b5)):
            x = jax.nn.relu(jnp.dot(x, kr[...],
                                    preferred_element_type=jnp.float32)
                            + br[...])
        x = jax.nn.relu(jnp.dot(x, k6[...],
                                preferred_element_type=jnp.float32) + b6[...])
        o_ref[...] = jnp.concatenate(
            [x, jnp.zeros((_HBP, 104), jnp.float32)], axis=1)

    grid = GHP // _HBP
    spec = pl.BlockSpec((_HBP, 128), lambda i: (i, 0))

    def wspec(w):
        return pl.BlockSpec(w.shape, lambda i: (0, 0))

    ws = [K1h, K1a] + Kmid
    return pl.pallas_call(
        body,
        grid=(grid,),
        in_specs=[spec, spec] + [wspec(w) for w in ws]
                 + [wspec(b) for b in bsp],
        out_specs=pl.BlockSpec((_HBP, 128), lambda i: (i, 0)),
        out_shape=jax.ShapeDtypeStruct((GHP, 128), jnp.float32),
    )(ghp, gap, *ws, *bsp)


def kernel(edge_index, emb, gconv_params, dense_params):
    row = edge_index[0].astype(jnp.int32)
    col = edge_index[1].astype(jnp.int32)

    rowp = jnp.concatenate([row, jnp.full((EP - E,), N, jnp.int32)])
    colp = jnp.concatenate([col, jnp.full((EP - E,), N, jnp.int32)])
    row2 = rowp.reshape(EP // ECHUNK, ECHUNK)
    col2 = colp.reshape(EP // ECHUNK, ECHUNK)

    embT = jnp.pad(emb.T, ((0, 0), (0, NP - N)))

    z16 = jnp.zeros((NPW, 16), jnp.float32)
    ones16 = jnp.ones((ECHUNK, 16), jnp.float32)
    deg2 = _make_deg_kernel()(row2, ones16, z16)
    deg2p = deg2.reshape(NC, NPP, 128)
    dinv = _tc_dinv(deg2p[0], deg2p[1])
    Cin = emb.shape[1]
    t0a, t0b, ua, ub = _make_pack_kernel(Cin)(embT, dinv.reshape(NP * 16))
    t0a = t0a.reshape(NPP, 128)
    t0b = t0b.reshape(NPP, 128)
    ua = ua.reshape(NPP, 128)
    ub = ub.reshape(NPP, 128)

    s_kernel = _make_s_kernel()

    def S(u_packed):
        s = s_kernel(u_packed.reshape(NP, 16), rowp, col2, z16)
        sp = s.reshape(NC, NPP, 128)
        return sp[0], sp[1]

    def run_layer(Ts_streams, us, p, w_slices):
        ns = len(us)
        for k in range(1, K):
            ss = [S(us[s]) for s in range(ns)]
            if k == 1:
                tprevs = [Ts_streams[s][0] for s in range(ns)]
                tks, us = _tc_cheb(ss, dinv, tprevs, -1.0, 0.0)
            else:
                tprevs = [Ts_streams[s][k - 2] for s in range(ns)]
                tks, us = _tc_cheb(ss, dinv, tprevs, -2.0, -1.0)
            for s in range(ns):
                Ts_streams[s].append(tks[s])
        Kz, Kh = [], []
        for s, sl in enumerate(w_slices):
            for k in range(K):
                Kz.append(_kron(8, sl(p["xz"]["W"][k])))
        for s, sl in enumerate(w_slices):
            for k in range(K):
                Kh.append(_kron(8, sl(p["xh"]["W"][k])))
        bz = jnp.tile(p["xz"]["b"] + p["hz"]["b"], 8)[None, :]
        bh = jnp.tile(p["xh"]["b"] + p["hh"]["b"], 8)[None, :]
        return _tc_gru(Ts_streams, Kz, Kh, bz, bh, dinv)

    slice_a = lambda w: w[:16, :]
    slice_b = lambda w: jnp.pad(w[16:, :], ((0, 12), (0, 0)))
    h, u = run_layer([[t0a], [t0b]], [ua, ub], gconv_params[0],
                     [slice_a, slice_b])

    for p in gconv_params[1:]:
        h, u = run_layer([[h]], [u], p, [lambda w: w])

    zpad = jnp.zeros((EH - E,), jnp.int32)
    idxh = jnp.concatenate([row, zpad, col, zpad])
    g3 = _make_gather_kernel()(h.reshape(NP, 16), idxh)
    gp = g3.reshape(2 * GHP, 128)

    Ws = [dp["W"] for dp in dense_params]
    bs = [dp["b"] for dp in dense_params]
    logits = _tc_head(gp[:GHP], gp[GHP:], Ws, bs)

    out3 = _make_softmax3_kernel()(logits.reshape(GHP * 128))
    return out3.T
